# Initial kernel scaffold; baseline (speedup 1.0000x reference)
import jax
import jax.numpy as jnp
from jax.experimental import pallas as pl


def kernel(hidden_states, W1, b1, W2, b2, Wc, bc, Wg1, bg1, Wg2, bg2, expert_scales, expert_biases):
    raise NotImplementedError("write your pallas kernel here")


# speedup vs baseline: 2.8138x; 2.8138x over previous
import jax
import jax.numpy as jnp
from jax import lax
from jax.experimental import pallas as pl
from jax.experimental.pallas import tpu as pltpu

_TN = 512


def _router_body(x_ref, w1_ref, b1_ref, w2_ref, b2_ref, sc_ref, bi_ref, o_ref):
    x = x_ref[...]
    h = jnp.maximum(
        lax.dot_general(x, w1_ref[...], (((1,), (0,)), ((), ())),
                        preferred_element_type=jnp.float32)
        + b1_ref[...][None, :], 0.0)
    s = lax.dot_general(h, w2_ref[...], (((1,), (0,)), ((), ())),
                        preferred_element_type=jnp.float32) + b2_ref[...][None, :]
    e = s.shape[1]
    m = jnp.max(s, axis=1, keepdims=True)
    iota = lax.broadcasted_iota(jnp.int32, s.shape, 1)
    idx = jnp.min(jnp.where(s == m, iota, e), axis=1, keepdims=True)
    oh = (iota == idx).astype(jnp.float32)
    ws = lax.dot_general(oh, sc_ref[...], (((1,), (0,)), ((), ())),
                         preferred_element_type=jnp.float32)
    wb = lax.dot_general(oh, bi_ref[...], (((1,), (0,)), ((), ())),
                         preferred_element_type=jnp.float32)
    o_ref[...] = x * ws + wb


def kernel(hidden_states, W1, b1, W2, b2, Wc, bc, Wg1, bg1, Wg2, bg2,
           expert_scales, expert_biases):
    B, S, H = hidden_states.shape
    N = B * S
    E, _ = expert_scales.shape
    RH = W1.shape[1]
    flat = hidden_states.reshape(N, H)
    out = pl.pallas_call(
        _router_body,
        grid=(N // _TN,),
        in_specs=[
            pl.BlockSpec((_TN, H), lambda i: (i, 0)),
            pl.BlockSpec((H, RH), lambda i: (0, 0)),
            pl.BlockSpec((RH,), lambda i: (0,)),
            pl.BlockSpec((RH, E), lambda i: (0, 0)),
            pl.BlockSpec((E,), lambda i: (0,)),
            pl.BlockSpec((E, H), lambda i: (0, 0)),
            pl.BlockSpec((E, H), lambda i: (0, 0)),
        ],
        out_specs=pl.BlockSpec((_TN, H), lambda i: (i, 0)),
        out_shape=jax.ShapeDtypeStruct((N, H), jnp.float32),
        compiler_params=pltpu.CompilerParams(
            dimension_semantics=("parallel",)),
    )(flat, W1, b1, W2, b2, expert_scales, expert_biases)
    return out.reshape(B, S, H)
